# Initial kernel scaffold; baseline (speedup 1.0000x reference)
import jax
import jax.numpy as jnp
from jax.experimental import pallas as pl


def kernel(V, E, K, edge_mask, params):
    raise NotImplementedError("write your pallas kernel here")


# speedup vs baseline: 8.1776x; 8.1776x over previous
import functools

import jax
import jax.numpy as jnp
from jax import lax
from jax.experimental import pallas as pl
from jax.experimental.pallas import tpu as pltpu
from jax.experimental.pallas import tpu_sc as plsc

F32 = jnp.float32
BF16 = jnp.bfloat16

_NC, _NS = 2, 16
_NW = _NC * _NS


def _sc_gather(table, idx):
    V, D = table.shape
    B = idx.shape[0]
    assert B % _NW == 0
    b_per_w = B // _NW
    C = 80
    assert b_per_w % C == 0
    n_chunks = b_per_w // C

    mesh = plsc.VectorSubcoreMesh(core_axis_name="c", subcore_axis_name="s")

    @functools.partial(
        pl.kernel,
        mesh=mesh,
        out_type=jax.ShapeDtypeStruct((B, D), F32),
        scratch_types=[
            pltpu.VMEM((C,), jnp.int32),
            pltpu.VMEM((C, D), F32),
            pltpu.SemaphoreType.DMA,
        ],
    )
    def k(table_hbm, idx_hbm, out_hbm, idx_v, rows_v, sem):
        wid = lax.axis_index("s") * _NC + lax.axis_index("c")

        def body(i, carry):
            base = wid * b_per_w + i * C
            pltpu.sync_copy(idx_hbm.at[pl.ds(base, C)], idx_v)
            pltpu.async_copy(table_hbm.at[idx_v], rows_v, sem).wait()
            pltpu.sync_copy(rows_v, out_hbm.at[pl.ds(base, C)])
            return carry

        lax.fori_loop(0, n_chunks, body, 0)

    return k(table, idx)


def _ln(x, g, b):
    m = jnp.mean(x, axis=-1, keepdims=True)
    v = jnp.mean((x - m) * (x - m), axis=-1, keepdims=True)
    return (x - m) * lax.rsqrt(v + 1e-5) * g + b


def _edge_mlp(nb_nodes, kn, hv_b, x_cat, wa, wbc, b1, w2, b2, w3, b3):
    d = hv_b.shape[-1]
    hvw = jnp.dot(hv_b, wa[:], preferred_element_type=F32)
    t = jnp.dot(x_cat, wbc[:], preferred_element_type=F32)
    t3 = t.reshape(nb_nodes, kn, d) + hvw[:, None, :] + b1[:][None]
    m1 = jax.nn.gelu(t3).reshape(nb_nodes * kn, d).astype(BF16)
    m2 = jnp.dot(m1, w2[:], preferred_element_type=F32) + b2[:]
    m2 = jax.nn.gelu(m2).astype(BF16)
    return jnp.dot(m2, w3[:], preferred_element_type=F32) + b3[:]


def _node_tail(hv, dh, wf1, bf1, wf2, bf2, g1, bn1, g2, bn2):
    h1 = _ln(hv + dh, g1[:], bn1[:])
    f1 = jnp.dot(h1.astype(BF16), wf1[:], preferred_element_type=F32) + bf1[:]
    f1 = jax.nn.gelu(f1).astype(BF16)
    ff = jnp.dot(f1, wf2[:], preferred_element_type=F32) + bf2[:]
    return _ln(h1 + ff, g2[:], bn2[:])


def _node_body(nb_nodes, kn,
               hv_ref, he_ref, nbr_ref, mask_ref,
               wa, wbc, b1, w2, b2, w3, b3,
               wf1, bf1, wf2, bf2, g1, bn1, g2, bn2,
               out_ref):
    d = hv_ref.shape[-1]
    hv = hv_ref[:]
    x_cat = jnp.concatenate(
        [he_ref[:].astype(BF16), nbr_ref[:].astype(BF16)], axis=-1)
    m3 = _edge_mlp(nb_nodes, kn, hv.astype(BF16), x_cat,
                   wa, wbc, b1, w2, b2, w3, b3)
    mask3 = mask_ref[:][:, :, None]
    dh = jnp.sum(m3.reshape(nb_nodes, kn, d) * mask3, axis=1) * (1.0 / kn)
    out_ref[:] = _node_tail(hv, dh, wf1, bf1, wf2, bf2, g1, bn1, g2, bn2)


def _fused_body(nb_nodes, kn,
                hv_ref, he_ref, nbr_ref, mask_ref,
                w11a, w11bc, b11, w12, b12, w13, b13, g3, bn3,
                wa, wbc, b1, w2, b2, w3, b3,
                wf1, bf1, wf2, bf2, g1, bn1, g2, bn2,
                he_out_ref, hv_out_ref):
    d = hv_ref.shape[-1]
    e = nb_nodes * kn
    hv = hv_ref[:]
    hv_b = hv.astype(BF16)
    nb_b = nbr_ref[:].astype(BF16)
    he = he_ref[:]
    mask3 = mask_ref[:][:, :, None]
    x_cat = jnp.concatenate([he.astype(BF16), nb_b], axis=-1)
    me3 = _edge_mlp(nb_nodes, kn, hv_b, x_cat,
                    w11a, w11bc, b11, w12, b12, w13, b13)
    he_new = _ln(he.reshape(nb_nodes, kn, d) + me3.reshape(nb_nodes, kn, d) * mask3,
                 g3[:], bn3[:])
    he_new = he_new.reshape(e, d)
    he_out_ref[:] = he_new
    x_cat2 = jnp.concatenate([he_new.astype(BF16), nb_b], axis=-1)
    m3 = _edge_mlp(nb_nodes, kn, hv_b, x_cat2, wa, wbc, b1, w2, b2, w3, b3)
    dh = jnp.sum(m3.reshape(nb_nodes, kn, d) * mask3, axis=1) * (1.0 / kn)
    hv_out_ref[:] = _node_tail(hv, dh, wf1, bf1, wf2, bf2, g1, bn1, g2, bn2)


def _edge_body(nb_nodes, kn,
               hv_ref, he_ref, nbr_ref, mask_ref,
               w11a, w11bc, b11, w12, b12, w13, b13, g3, bn3,
               he_out_ref):
    d = hv_ref.shape[-1]
    hv = hv_ref[:]
    he = he_ref[:]
    x_cat = jnp.concatenate([he.astype(BF16), nbr_ref[:].astype(BF16)], axis=-1)
    me3 = _edge_mlp(nb_nodes, kn, hv.astype(BF16), x_cat,
                    w11a, w11bc, b11, w12, b12, w13, b13)
    mask3 = mask_ref[:][:, :, None]
    he_new = _ln(he.reshape(nb_nodes, kn, d) + me3.reshape(nb_nodes, kn, d) * mask3,
                 g3[:], bn3[:])
    he_out_ref[:] = he_new.reshape(nb_nodes * kn, d)


def _specs(nb_nodes, kn, d, weight_shapes):
    sp = [
        pl.BlockSpec((nb_nodes, d), lambda i: (i, 0)),
        pl.BlockSpec((nb_nodes * kn, d), lambda i: (i, 0)),
        pl.BlockSpec((nb_nodes * kn, d), lambda i: (i, 0)),
        pl.BlockSpec((nb_nodes, kn), lambda i: (i, 0)),
    ]
    for s in weight_shapes:
        sp.append(pl.BlockSpec(s, (lambda r: (lambda i: (0,) * r))(len(s))))
    return sp


def _call_node(hv, he, nbr, mask, wts, nb_nodes):
    n, d = hv.shape
    kn = mask.shape[1]
    grid = (n // nb_nodes,)
    return pl.pallas_call(
        functools.partial(_node_body, nb_nodes, kn),
        grid=grid,
        in_specs=_specs(nb_nodes, kn, d, [w.shape for w in wts]),
        out_specs=pl.BlockSpec((nb_nodes, d), lambda i: (i, 0)),
        out_shape=jax.ShapeDtypeStruct((n, d), F32),
    )(hv, he, nbr, mask, *wts)


def _call_fused(hv, he, nbr, mask, wts, nb_nodes):
    n, d = hv.shape
    kn = mask.shape[1]
    grid = (n // nb_nodes,)
    return pl.pallas_call(
        functools.partial(_fused_body, nb_nodes, kn),
        grid=grid,
        in_specs=_specs(nb_nodes, kn, d, [w.shape for w in wts]),
        out_specs=[
            pl.BlockSpec((nb_nodes * kn, d), lambda i: (i, 0)),
            pl.BlockSpec((nb_nodes, d), lambda i: (i, 0)),
        ],
        out_shape=[
            jax.ShapeDtypeStruct((n * kn, d), F32),
            jax.ShapeDtypeStruct((n, d), F32),
        ],
    )(hv, he, nbr, mask, *wts)


def _call_edge(hv, he, nbr, mask, wts, nb_nodes):
    n, d = hv.shape
    kn = mask.shape[1]
    grid = (n // nb_nodes,)
    return pl.pallas_call(
        functools.partial(_edge_body, nb_nodes, kn),
        grid=grid,
        in_specs=_specs(nb_nodes, kn, d, [w.shape for w in wts]),
        out_specs=pl.BlockSpec((nb_nodes * kn, d), lambda i: (i, 0)),
        out_shape=jax.ShapeDtypeStruct((n * kn, d), F32),
    )(hv, he, nbr, mask, *wts)


def _layer_weights(p, l):
    d = p["W2"].shape[-1]

    def row(x):
        return x[l][None, :]

    node = (
        p["W1"][l, :d].astype(BF16), p["W1"][l, d:].astype(BF16), row(p["b1"]),
        p["W2"][l].astype(BF16), row(p["b2"]),
        p["W3"][l].astype(BF16), row(p["b3"]),
        p["Wf1"][l].astype(BF16), row(p["bf1"]),
        p["Wf2"][l].astype(BF16), row(p["bf2"]),
        row(p["g1"]), row(p["bn1"]), row(p["g2"]), row(p["bn2"]),
    )
    edge = (
        p["W11"][l, :d].astype(BF16), p["W11"][l, d:].astype(BF16), row(p["b11"]),
        p["W12"][l].astype(BF16), row(p["b12"]),
        p["W13"][l].astype(BF16), row(p["b13"]),
        row(p["g3"]), row(p["bn3"]),
    )
    return node, edge


def kernel(V, E, K, edge_mask, params):
    b, n, kn, d = E.shape
    num_layers = params["W1"].shape[0]
    assert b == 1
    hv = V[0]
    he = E[0].reshape(n * kn, d)
    idx = K[0].reshape(n * kn)
    mask = edge_mask[0]
    nb_nodes = 200
    assert n % nb_nodes == 0

    wts = [_layer_weights(params, l) for l in range(num_layers)]

    nbr = _sc_gather(hv, idx)
    hv = _call_node(hv, he, nbr, mask, wts[0][0], nb_nodes)
    for l in range(num_layers - 1):
        nbr = _sc_gather(hv, idx)
        he, hv = _call_fused(hv, he, nbr, mask,
                             wts[l][1] + wts[l + 1][0], nb_nodes)
    nbr = _sc_gather(hv, idx)
    he = _call_edge(hv, he, nbr, mask, wts[num_layers - 1][1], nb_nodes)

    return (hv[None], he.reshape(1, n, kn, d))


# speedup vs baseline: 10.1872x; 1.2457x over previous
import functools

import jax
import jax.numpy as jnp
from jax import lax
from jax.experimental import pallas as pl
from jax.experimental.pallas import tpu as pltpu
from jax.experimental.pallas import tpu_sc as plsc

F32 = jnp.float32
BF16 = jnp.bfloat16

_NC, _NS = 2, 16
_NW = _NC * _NS


def _pack_rows(x):
    n, d = x.shape
    u = lax.bitcast_convert_type(x.astype(BF16), jnp.uint16).reshape(n, d // 2, 2)
    w = u[..., 0].astype(jnp.uint32) | (u[..., 1].astype(jnp.uint32) << 16)
    return lax.bitcast_convert_type(w, jnp.int32)


_NBUF = 5
_LAG = 3


def _sc_gather(table, idx3):
    _, dw = table.shape
    nw, n_chunks, C = idx3.shape
    assert nw == _NW and n_chunks % _NBUF == 0
    b_per_w = n_chunks * C
    B = nw * b_per_w

    mesh = plsc.VectorSubcoreMesh(core_axis_name="c", subcore_axis_name="s")

    @functools.partial(
        pl.kernel,
        mesh=mesh,
        out_type=jax.ShapeDtypeStruct((B, dw), F32),
        scratch_types=(
            [pltpu.VMEM((n_chunks, C), jnp.int32)]
            + [pltpu.VMEM((C, dw), F32) for _ in range(_NBUF)]
            + [pltpu.SemaphoreType.DMA for _ in range(2 * _NBUF)]
        ),
    )
    def k(table_hbm, idx_hbm, out_hbm, idx_all, *bufs_sems):
        rows = bufs_sems[:_NBUF]
        sg = bufs_sems[_NBUF:2 * _NBUF]
        sw = bufs_sems[2 * _NBUF:]
        wid = lax.axis_index("s") * _NC + lax.axis_index("c")
        base = wid * b_per_w

        pltpu.sync_copy(idx_hbm.at[wid], idx_all)

        def gather_copy(c, b):
            return pltpu.make_async_copy(
                table_hbm.at[idx_all.at[c]], rows[b], sg[b])

        def wb_copy(c, b):
            return pltpu.make_async_copy(
                rows[b], out_hbm.at[pl.ds(base + c * C, C)], sw[b])

        for b in range(_LAG):
            gather_copy(b, b).start()

        def body(i, carry):
            for b in range(_NBUF):
                p = i * _NBUF + b
                cg = p + _LAG
                bg = (b + _LAG) % _NBUF

                @pl.when(cg < n_chunks)
                def _():
                    @pl.when(cg >= _NBUF)
                    def _():
                        wb_copy(cg - _NBUF, bg).wait()
                    gather_copy(cg, bg).start()

                gather_copy(p, b).wait()
                wb_copy(p, b).start()
            return carry

        lax.fori_loop(0, n_chunks // _NBUF, body, 0)

        for b in range(_NBUF):
            c = n_chunks - _NBUF + b
            wb_copy(c, c % _NBUF).wait()

    return k(table, idx3)


def _ln(x, g, b):
    m = jnp.mean(x, axis=-1, keepdims=True)
    v = jnp.mean((x - m) * (x - m), axis=-1, keepdims=True)
    return (x - m) * lax.rsqrt(v + 1e-5) * g + b


def _edge_mlp(nb_nodes, kn, hv_b, x_cat, wa, wbc, b1, w2, b2, w3, b3):
    d = hv_b.shape[-1]
    hvw = jnp.dot(hv_b, wa[:], preferred_element_type=F32)
    t = jnp.dot(x_cat, wbc[:], preferred_element_type=F32)
    t3 = t.reshape(nb_nodes, kn, d) + hvw[:, None, :] + b1[:][None]
    m1 = jax.nn.gelu(t3).reshape(nb_nodes * kn, d).astype(BF16)
    m2 = jnp.dot(m1, w2[:], preferred_element_type=F32) + b2[:]
    m2 = jax.nn.gelu(m2).astype(BF16)
    return jnp.dot(m2, w3[:], preferred_element_type=F32) + b3[:]


def _node_tail(hv, dh, wf1, bf1, wf2, bf2, g1, bn1, g2, bn2):
    h1 = _ln(hv + dh, g1[:], bn1[:])
    f1 = jnp.dot(h1.astype(BF16), wf1[:], preferred_element_type=F32) + bf1[:]
    f1 = jax.nn.gelu(f1).astype(BF16)
    ff = jnp.dot(f1, wf2[:], preferred_element_type=F32) + bf2[:]
    return _ln(h1 + ff, g2[:], bn2[:])


def _unpack_cat(he_bf, nb_f32):
    return jnp.concatenate([he_bf, nb_f32.astype(BF16)], axis=-1)


def _node_body(nb_nodes, kn,
               hv_ref, he_ref, nbr_ref, mask_ref,
               wa, wbc, b1, w2, b2, w3, b3,
               wf1, bf1, wf2, bf2, g1, bn1, g2, bn2,
               out_ref):
    d = hv_ref.shape[-1]
    hv = hv_ref[:]
    x_cat = _unpack_cat(he_ref[:].astype(BF16), nbr_ref[:])
    m3 = _edge_mlp(nb_nodes, kn, hv.astype(BF16), x_cat,
                   wa, wbc, b1, w2, b2, w3, b3)
    mask3 = mask_ref[:][:, :, None]
    dh = jnp.sum(m3.reshape(nb_nodes, kn, d) * mask3, axis=1) * (1.0 / kn)
    out_ref[:] = _node_tail(hv, dh, wf1, bf1, wf2, bf2, g1, bn1, g2, bn2)


def _fused_body(nb_nodes, kn,
                hv_ref, he_ref, nbr_ref, mask_ref,
                w11a, w11bc, b11, w12, b12, w13, b13, g3, bn3,
                wa, wbc, b1, w2, b2, w3, b3,
                wf1, bf1, wf2, bf2, g1, bn1, g2, bn2,
                he_out_ref, hv_out_ref):
    d = hv_ref.shape[-1]
    e = nb_nodes * kn
    hv = hv_ref[:]
    hv_b = hv.astype(BF16)
    nb_p = nbr_ref[:]
    he = he_ref[:]
    mask3 = mask_ref[:][:, :, None]
    x_cat = _unpack_cat(he.astype(BF16), nb_p)
    me3 = _edge_mlp(nb_nodes, kn, hv_b, x_cat,
                    w11a, w11bc, b11, w12, b12, w13, b13)
    he_new = _ln(he.reshape(nb_nodes, kn, d) + me3.reshape(nb_nodes, kn, d) * mask3,
                 g3[:], bn3[:])
    he_new = he_new.reshape(e, d)
    he_out_ref[:] = he_new
    x_cat2 = jnp.concatenate([he_new.astype(BF16), x_cat[:, d:]], axis=-1)
    m3 = _edge_mlp(nb_nodes, kn, hv_b, x_cat2, wa, wbc, b1, w2, b2, w3, b3)
    dh = jnp.sum(m3.reshape(nb_nodes, kn, d) * mask3, axis=1) * (1.0 / kn)
    hv_out_ref[:] = _node_tail(hv, dh, wf1, bf1, wf2, bf2, g1, bn1, g2, bn2)


def _edge_body(nb_nodes, kn,
               hv_ref, he_ref, nbr_ref, mask_ref,
               w11a, w11bc, b11, w12, b12, w13, b13, g3, bn3,
               he_out_ref):
    d = hv_ref.shape[-1]
    hv = hv_ref[:]
    he = he_ref[:]
    x_cat = _unpack_cat(he.astype(BF16), nbr_ref[:])
    me3 = _edge_mlp(nb_nodes, kn, hv.astype(BF16), x_cat,
                    w11a, w11bc, b11, w12, b12, w13, b13)
    mask3 = mask_ref[:][:, :, None]
    he_new = _ln(he.reshape(nb_nodes, kn, d) + me3.reshape(nb_nodes, kn, d) * mask3,
                 g3[:], bn3[:])
    he_out_ref[:] = he_new.reshape(nb_nodes * kn, d)


def _specs(nb_nodes, kn, d, weight_shapes):
    sp = [
        pl.BlockSpec((nb_nodes, d), lambda i: (i, 0)),
        pl.BlockSpec((nb_nodes * kn, d), lambda i: (i, 0)),
        pl.BlockSpec((nb_nodes * kn, d), lambda i: (i, 0)),
        pl.BlockSpec((nb_nodes, kn), lambda i: (i, 0)),
    ]
    for s in weight_shapes:
        sp.append(pl.BlockSpec(s, (lambda r: (lambda i: (0,) * r))(len(s))))
    return sp


def _call_node(hv, he, nbr, mask, wts, nb_nodes):
    n, d = hv.shape
    kn = mask.shape[1]
    grid = (n // nb_nodes,)
    return pl.pallas_call(
        functools.partial(_node_body, nb_nodes, kn),
        grid=grid,
        in_specs=_specs(nb_nodes, kn, d, [w.shape for w in wts]),
        out_specs=pl.BlockSpec((nb_nodes, d), lambda i: (i, 0)),
        out_shape=jax.ShapeDtypeStruct((n, d), F32),
    )(hv, he, nbr, mask, *wts)


def _call_fused(hv, he, nbr, mask, wts, nb_nodes):
    n, d = hv.shape
    kn = mask.shape[1]
    grid = (n // nb_nodes,)
    return pl.pallas_call(
        functools.partial(_fused_body, nb_nodes, kn),
        grid=grid,
        in_specs=_specs(nb_nodes, kn, d, [w.shape for w in wts]),
        out_specs=[
            pl.BlockSpec((nb_nodes * kn, d), lambda i: (i, 0)),
            pl.BlockSpec((nb_nodes, d), lambda i: (i, 0)),
        ],
        out_shape=[
            jax.ShapeDtypeStruct((n * kn, d), F32),
            jax.ShapeDtypeStruct((n, d), F32),
        ],
    )(hv, he, nbr, mask, *wts)


def _call_edge(hv, he, nbr, mask, wts, nb_nodes):
    n, d = hv.shape
    kn = mask.shape[1]
    grid = (n // nb_nodes,)
    return pl.pallas_call(
        functools.partial(_edge_body, nb_nodes, kn),
        grid=grid,
        in_specs=_specs(nb_nodes, kn, d, [w.shape for w in wts]),
        out_specs=pl.BlockSpec((nb_nodes * kn, d), lambda i: (i, 0)),
        out_shape=jax.ShapeDtypeStruct((n * kn, d), F32),
    )(hv, he, nbr, mask, *wts)


def _layer_weights(p, l):
    d = p["W2"].shape[-1]

    def row(x):
        return x[l][None, :]

    def wbc(w):
        return w[l, d:].astype(BF16)

    node = (
        p["W1"][l, :d].astype(BF16), wbc(p["W1"]), row(p["b1"]),
        p["W2"][l].astype(BF16), row(p["b2"]),
        p["W3"][l].astype(BF16), row(p["b3"]),
        p["Wf1"][l].astype(BF16), row(p["bf1"]),
        p["Wf2"][l].astype(BF16), row(p["bf2"]),
        row(p["g1"]), row(p["bn1"]), row(p["g2"]), row(p["bn2"]),
    )
    edge = (
        p["W11"][l, :d].astype(BF16), wbc(p["W11"]), row(p["b11"]),
        p["W12"][l].astype(BF16), row(p["b12"]),
        p["W13"][l].astype(BF16), row(p["b13"]),
        row(p["g3"]), row(p["bn3"]),
    )
    return node, edge


def kernel(V, E, K, edge_mask, params):
    b, n, kn, d = E.shape
    num_layers = params["W1"].shape[0]
    assert b == 1
    hv = V[0]
    he = E[0].reshape(n * kn, d)
    C = 80
    idx3 = K[0].reshape(_NW, n * kn // (_NW * C), C)
    mask = edge_mask[0]
    nb_nodes = 200
    assert n % nb_nodes == 0

    wts = [_layer_weights(params, l) for l in range(num_layers)]

    nbr = _sc_gather(hv, idx3)
    hv = _call_node(hv, he, nbr, mask, wts[0][0], nb_nodes)
    for l in range(num_layers - 1):
        nbr = _sc_gather(hv, idx3)
        he, hv = _call_fused(hv, he, nbr, mask,
                             wts[l][1] + wts[l + 1][0], nb_nodes)
    nbr = _sc_gather(hv, idx3)
    he = _call_edge(hv, he, nbr, mask, wts[num_layers - 1][1], nb_nodes)

    return (hv[None], he.reshape(1, n, kn, d))
